# baseline (speedup 1.0000x reference)
import numpy as np

import jax
import jax.numpy as jnp
from jax.experimental import pallas as pl
from jax.experimental.pallas import tpu as pltpu

_B, _N, _D, _E, _L, _H = 16, 512, 128, 128, 3, 4
_DH = _E // _H
_MASK_RATIO = 0.15


def _body(x_ref, A_ref, Ap_ref, m_ref, mt_ref, Win_ref, bin_ref,
          Wq_ref, Wk_ref, Wv_ref, Wo_ref, W1_ref, b1_ref, W2_ref, b2_ref,
          pred_ref, num_ref):
    x = x_ref[0]
    m = m_ref[0]
    xf = x * (1.0 - m) + m * mt_ref[...]
    h = jnp.dot(xf, Win_ref[...], preferred_element_type=jnp.float32) + bin_ref[...]
    bias = A_ref[0] + Ap_ref[0]
    bias = bias - jnp.max(bias, axis=-1, keepdims=True)
    scale = 1.0 / np.sqrt(_DH)
    for l in range(_L):
        q = jnp.dot(h, Wq_ref[l], preferred_element_type=jnp.float32) * scale
        k = jnp.dot(h, Wk_ref[l], preferred_element_type=jnp.float32)
        v = jnp.dot(h, Wv_ref[l], preferred_element_type=jnp.float32)
        ones_col = jnp.ones((v.shape[0], 1), jnp.float32)
        o_heads = []
        for hh in range(_H):
            sl = slice(hh * _DH, (hh + 1) * _DH)
            qh, kh = q[:, sl].astype(jnp.bfloat16), k[:, sl].astype(jnp.bfloat16)
            vh1 = jnp.concatenate([v[:, sl], ones_col], axis=1)
            logits = jax.lax.dot_general(
                qh, kh, (((1,), (1,)), ((), ())),
                preferred_element_type=jnp.float32) + bias
            e = jnp.exp(logits)
            t = jnp.dot(e.astype(jnp.bfloat16), vh1.astype(jnp.bfloat16),
                        preferred_element_type=jnp.float32)
            o_heads.append(t[:, :_DH] * (1.0 / t[:, _DH:_DH + 1]))
        o = jnp.concatenate(o_heads, axis=1)
        h = jnp.maximum(h + jnp.dot(o, Wo_ref[l], preferred_element_type=jnp.float32), 0.0)
    hid = jnp.maximum(jnp.dot(h, W1_ref[...], preferred_element_type=jnp.float32)
                      + b1_ref[...], 0.0)
    pred = jnp.dot(hid, W2_ref[...], preferred_element_type=jnp.float32) + b2_ref[...]
    pred_ref[0] = pred
    lp = jnp.mean((pred - x) ** 2, axis=-1, keepdims=True)
    num_ref[0] = jnp.sum(lp * m, axis=0, keepdims=True)


def kernel(x, A, A_phi, mask_token, W_in, b_in, Wq, Wk, Wv, Wo, W1, b1, W2, b2):
    B, N, D = x.shape
    E = W_in.shape[1]
    len_keep = int(N * (1.0 - _MASK_RATIO))
    noise = jax.random.uniform(jax.random.key(42), (B, N), dtype=jnp.float32)
    ids_shuffle = jnp.argsort(noise, axis=1)
    ids_restore = jnp.argsort(ids_shuffle, axis=1)
    mask = jnp.ones((B, N), dtype=jnp.float32).at[:, :len_keep].set(0.0)
    mask = jnp.take_along_axis(mask, ids_restore, axis=1)

    m3 = mask.reshape(B, N, 1)
    mt2 = mask_token.reshape(1, D)
    bin2 = b_in.reshape(1, E)
    b12 = b1.reshape(1, -1)
    b22 = b2.reshape(1, D)

    grid = (B,)
    pred, num = pl.pallas_call(
        _body,
        grid=grid,
        in_specs=[
            pl.BlockSpec((1, N, D), lambda b: (b, 0, 0)),
            pl.BlockSpec((1, N, N), lambda b: (b, 0, 0)),
            pl.BlockSpec((1, N, N), lambda b: (b, 0, 0)),
            pl.BlockSpec((1, N, 1), lambda b: (b, 0, 0)),
            pl.BlockSpec((1, D), lambda b: (0, 0)),
            pl.BlockSpec((D, E), lambda b: (0, 0)),
            pl.BlockSpec((1, E), lambda b: (0, 0)),
            pl.BlockSpec((_L, E, E), lambda b: (0, 0, 0)),
            pl.BlockSpec((_L, E, E), lambda b: (0, 0, 0)),
            pl.BlockSpec((_L, E, E), lambda b: (0, 0, 0)),
            pl.BlockSpec((_L, E, E), lambda b: (0, 0, 0)),
            pl.BlockSpec((E, 2 * E), lambda b: (0, 0)),
            pl.BlockSpec((1, 2 * E), lambda b: (0, 0)),
            pl.BlockSpec((2 * E, D), lambda b: (0, 0)),
            pl.BlockSpec((1, D), lambda b: (0, 0)),
        ],
        out_specs=[
            pl.BlockSpec((1, N, D), lambda b: (b, 0, 0)),
            pl.BlockSpec((1, 1, 1), lambda b: (b, 0, 0)),
        ],
        out_shape=[
            jax.ShapeDtypeStruct((B, N, D), jnp.float32),
            jax.ShapeDtypeStruct((B, 1, 1), jnp.float32),
        ],
        compiler_params=pltpu.CompilerParams(
            dimension_semantics=("arbitrary",)),
    )(x, A, A_phi, m3, mt2, W_in, bin2, Wq, Wk, Wv, Wo, W1, b12, W2, b22)

    loss = jnp.sum(num) / jnp.sum(mask)
    return pred, loss, mask


# speedup vs baseline: 1.0015x; 1.0015x over previous
import numpy as np

import jax
import jax.numpy as jnp
from jax.experimental import pallas as pl
from jax.experimental.pallas import tpu as pltpu

_B, _N, _D, _E, _L, _H = 16, 512, 128, 128, 3, 4
_DH = _E // _H
_MASK_RATIO = 0.15


def _body(x_ref, A_ref, Ap_ref, m_ref, mt_ref, Win_ref, bin_ref,
          Wq_ref, Wk_ref, Wv_ref, Wo_ref, W1_ref, b1_ref, W2_ref, b2_ref,
          pred_ref, num_ref):
    x = x_ref[0]
    m = m_ref[0]
    xf = x * (1.0 - m) + m * mt_ref[...]
    h = jnp.dot(xf, Win_ref[...], preferred_element_type=jnp.float32) + bin_ref[...]
    bias = A_ref[0] + Ap_ref[0]
    bias = bias - jnp.max(bias, axis=-1, keepdims=True)
    eb = jnp.exp(bias)
    scale = 1.0 / np.sqrt(_DH)
    for l in range(_L):
        q = jnp.dot(h, Wq_ref[l], preferred_element_type=jnp.float32) * scale
        k = jnp.dot(h, Wk_ref[l], preferred_element_type=jnp.float32)
        v = jnp.dot(h, Wv_ref[l], preferred_element_type=jnp.float32)
        ones_col = jnp.ones((v.shape[0], 1), jnp.float32)
        o_heads = []
        for hh in range(_H):
            sl = slice(hh * _DH, (hh + 1) * _DH)
            qh, kh = q[:, sl], k[:, sl]
            vh1 = jnp.concatenate([v[:, sl], ones_col], axis=1)
            qk = jax.lax.dot_general(
                qh, kh, (((1,), (1,)), ((), ())),
                preferred_element_type=jnp.float32)
            e = eb * (1.0 + qk * (1.0 + 0.5 * qk))
            t = jnp.dot(e, vh1, preferred_element_type=jnp.float32)
            o_heads.append(t[:, :_DH] * (1.0 / t[:, _DH:_DH + 1]))
        o = jnp.concatenate(o_heads, axis=1)
        h = jnp.maximum(h + jnp.dot(o, Wo_ref[l], preferred_element_type=jnp.float32), 0.0)
    hid = jnp.maximum(jnp.dot(h, W1_ref[...], preferred_element_type=jnp.float32)
                      + b1_ref[...], 0.0)
    pred = jnp.dot(hid, W2_ref[...], preferred_element_type=jnp.float32) + b2_ref[...]
    pred_ref[0] = pred
    lp = jnp.mean((pred - x) ** 2, axis=-1, keepdims=True)
    num_ref[0] = jnp.sum(lp * m, axis=0, keepdims=True)


def kernel(x, A, A_phi, mask_token, W_in, b_in, Wq, Wk, Wv, Wo, W1, b1, W2, b2):
    B, N, D = x.shape
    E = W_in.shape[1]
    len_keep = int(N * (1.0 - _MASK_RATIO))
    noise = jax.random.uniform(jax.random.key(42), (B, N), dtype=jnp.float32)
    ids_shuffle = jnp.argsort(noise, axis=1)
    ids_restore = jnp.argsort(ids_shuffle, axis=1)
    mask = jnp.ones((B, N), dtype=jnp.float32).at[:, :len_keep].set(0.0)
    mask = jnp.take_along_axis(mask, ids_restore, axis=1)

    m3 = mask.reshape(B, N, 1)
    mt2 = mask_token.reshape(1, D)
    bin2 = b_in.reshape(1, E)
    b12 = b1.reshape(1, -1)
    b22 = b2.reshape(1, D)

    grid = (B,)
    pred, num = pl.pallas_call(
        _body,
        grid=grid,
        in_specs=[
            pl.BlockSpec((1, N, D), lambda b: (b, 0, 0)),
            pl.BlockSpec((1, N, N), lambda b: (b, 0, 0)),
            pl.BlockSpec((1, N, N), lambda b: (b, 0, 0)),
            pl.BlockSpec((1, N, 1), lambda b: (b, 0, 0)),
            pl.BlockSpec((1, D), lambda b: (0, 0)),
            pl.BlockSpec((D, E), lambda b: (0, 0)),
            pl.BlockSpec((1, E), lambda b: (0, 0)),
            pl.BlockSpec((_L, E, E), lambda b: (0, 0, 0)),
            pl.BlockSpec((_L, E, E), lambda b: (0, 0, 0)),
            pl.BlockSpec((_L, E, E), lambda b: (0, 0, 0)),
            pl.BlockSpec((_L, E, E), lambda b: (0, 0, 0)),
            pl.BlockSpec((E, 2 * E), lambda b: (0, 0)),
            pl.BlockSpec((1, 2 * E), lambda b: (0, 0)),
            pl.BlockSpec((2 * E, D), lambda b: (0, 0)),
            pl.BlockSpec((1, D), lambda b: (0, 0)),
        ],
        out_specs=[
            pl.BlockSpec((1, N, D), lambda b: (b, 0, 0)),
            pl.BlockSpec((1, 1, 1), lambda b: (b, 0, 0)),
        ],
        out_shape=[
            jax.ShapeDtypeStruct((B, N, D), jnp.float32),
            jax.ShapeDtypeStruct((B, 1, 1), jnp.float32),
        ],
        compiler_params=pltpu.CompilerParams(
            dimension_semantics=("arbitrary",)),
    )(x, A, A_phi, m3, mt2, W_in, bin2, Wq, Wk, Wv, Wo, W1, b12, W2, b22)

    loss = jnp.sum(num) / jnp.sum(mask)
    return pred, loss, mask


# speedup vs baseline: 1.0114x; 1.0098x over previous
import numpy as np

import jax
import jax.numpy as jnp
from jax.experimental import pallas as pl
from jax.experimental.pallas import tpu as pltpu

_B, _N, _D, _E, _L, _H = 16, 512, 128, 128, 3, 4
_DH = _E // _H
_MASK_RATIO = 0.15


def _body(x_ref, A_ref, Ap_ref, m_ref, mt_ref, Win_ref, bin_ref,
          Wq_ref, Wk_ref, Wv_ref, Wo_ref, W1_ref, b1_ref, W2_ref, b2_ref,
          pred_ref, num_ref):
    x = x_ref[0]
    m = m_ref[0]
    xf = x * (1.0 - m) + m * mt_ref[...]
    h = jnp.dot(xf, Win_ref[...], preferred_element_type=jnp.float32) + bin_ref[...]
    bias = A_ref[0] + Ap_ref[0]
    bias = bias - jnp.max(bias, axis=-1, keepdims=True)
    scale = 1.0 / np.sqrt(_DH)
    for l in range(_L):
        q = jnp.dot(h, Wq_ref[l], preferred_element_type=jnp.float32) * scale
        k = jnp.dot(h, Wk_ref[l], preferred_element_type=jnp.float32)
        v = jnp.dot(h, Wv_ref[l], preferred_element_type=jnp.float32)
        ones_col = jnp.ones((v.shape[0], 1), jnp.float32)
        o_heads = []
        for hh in range(_H):
            sl = slice(hh * _DH, (hh + 1) * _DH)
            qh, kh = q[:, sl], k[:, sl]
            vh1 = jnp.concatenate([v[:, sl], ones_col], axis=1)
            logits = jax.lax.dot_general(
                qh, kh, (((1,), (1,)), ((), ())),
                preferred_element_type=jnp.float32) + bias
            e = jnp.exp(logits)
            t = jnp.dot(e, vh1, preferred_element_type=jnp.float32)
            o_heads.append(t[:, :_DH] * (1.0 / t[:, _DH:_DH + 1]))
        o = jnp.concatenate(o_heads, axis=1)
        h = jnp.maximum(h + jnp.dot(o, Wo_ref[l], preferred_element_type=jnp.float32), 0.0)
    hid = jnp.maximum(jnp.dot(h, W1_ref[...], preferred_element_type=jnp.float32)
                      + b1_ref[...], 0.0)
    pred = jnp.dot(hid, W2_ref[...], preferred_element_type=jnp.float32) + b2_ref[...]
    pred_ref[0] = pred
    lp = jnp.mean((pred - x) ** 2, axis=-1, keepdims=True)
    num_ref[0] = jnp.sum(lp * m, axis=0, keepdims=True)


def kernel(x, A, A_phi, mask_token, W_in, b_in, Wq, Wk, Wv, Wo, W1, b1, W2, b2):
    B, N, D = x.shape
    E = W_in.shape[1]
    len_keep = int(N * (1.0 - _MASK_RATIO))
    noise = jax.random.uniform(jax.random.key(42), (B, N), dtype=jnp.float32)
    ids_shuffle = jnp.argsort(noise, axis=1)
    ids_restore = jnp.argsort(ids_shuffle, axis=1)
    mask = jnp.ones((B, N), dtype=jnp.float32).at[:, :len_keep].set(0.0)
    mask = jnp.take_along_axis(mask, ids_restore, axis=1)

    m3 = mask.reshape(B, N, 1)
    mt2 = mask_token.reshape(1, D)
    bin2 = b_in.reshape(1, E)
    b12 = b1.reshape(1, -1)
    b22 = b2.reshape(1, D)

    grid = (B,)
    pred, num = pl.pallas_call(
        _body,
        grid=grid,
        in_specs=[
            pl.BlockSpec((1, N, D), lambda b: (b, 0, 0)),
            pl.BlockSpec((1, N, N), lambda b: (b, 0, 0)),
            pl.BlockSpec((1, N, N), lambda b: (b, 0, 0)),
            pl.BlockSpec((1, N, 1), lambda b: (b, 0, 0)),
            pl.BlockSpec((1, D), lambda b: (0, 0)),
            pl.BlockSpec((D, E), lambda b: (0, 0)),
            pl.BlockSpec((1, E), lambda b: (0, 0)),
            pl.BlockSpec((_L, E, E), lambda b: (0, 0, 0)),
            pl.BlockSpec((_L, E, E), lambda b: (0, 0, 0)),
            pl.BlockSpec((_L, E, E), lambda b: (0, 0, 0)),
            pl.BlockSpec((_L, E, E), lambda b: (0, 0, 0)),
            pl.BlockSpec((E, 2 * E), lambda b: (0, 0)),
            pl.BlockSpec((1, 2 * E), lambda b: (0, 0)),
            pl.BlockSpec((2 * E, D), lambda b: (0, 0)),
            pl.BlockSpec((1, D), lambda b: (0, 0)),
        ],
        out_specs=[
            pl.BlockSpec((1, N, D), lambda b: (b, 0, 0)),
            pl.BlockSpec((1, 1, 1), lambda b: (b, 0, 0)),
        ],
        out_shape=[
            jax.ShapeDtypeStruct((B, N, D), jnp.float32),
            jax.ShapeDtypeStruct((B, 1, 1), jnp.float32),
        ],
        compiler_params=pltpu.CompilerParams(
            dimension_semantics=("arbitrary",)),
    )(x, A, A_phi, m3, mt2, W_in, bin2, Wq, Wk, Wv, Wo, W1, b12, W2, b22)

    loss = jnp.sum(num) / jnp.sum(mask)
    return pred, loss, mask


# speedup vs baseline: 1.3495x; 1.3343x over previous
import numpy as np

import jax
import jax.numpy as jnp
from jax.experimental import pallas as pl
from jax.experimental.pallas import tpu as pltpu

_B, _N, _D, _E, _L, _H = 16, 512, 128, 128, 3, 4
_DH = _E // _H
_MASK_RATIO = 0.15


def _body(x_ref, A_ref, Ap_ref, m_ref, mt_ref, Win_ref, bin_ref,
          Wq_ref, Wk_ref, Wv_ref, Wo_ref, W1_ref, b1_ref, W2_ref, b2_ref,
          pred_ref, num_ref):
    x = x_ref[0]
    m = m_ref[0]
    xf = x * (1.0 - m) + m * mt_ref[...]
    h = jnp.dot(xf, Win_ref[...], preferred_element_type=jnp.float32) + bin_ref[...]
    bias = A_ref[0] + Ap_ref[0]
    bias = bias - jnp.max(bias, axis=-1, keepdims=True)
    scale = 1.0 / np.sqrt(_DH)
    for l in range(_L):
        q = jnp.dot(h, Wq_ref[l], preferred_element_type=jnp.float32) * scale
        k = jnp.dot(h, Wk_ref[l], preferred_element_type=jnp.float32)
        v = jnp.dot(h, Wv_ref[l], preferred_element_type=jnp.float32)
        ones_col = jnp.ones((v.shape[0], 1), jnp.float32)
        o_heads = []
        for hh in range(_H):
            sl = slice(hh * _DH, (hh + 1) * _DH)
            qh, kh = q[:, sl], k[:, sl]
            vh1 = jnp.concatenate([v[:, sl], ones_col], axis=1)
            logits = jax.lax.dot_general(
                qh, kh, (((1,), (1,)), ((), ())),
                preferred_element_type=jnp.float32) + bias
            e = jnp.exp(logits)
            t = jnp.dot(e, vh1, preferred_element_type=jnp.float32)
            o_heads.append(t[:, :_DH] * (1.0 / t[:, _DH:_DH + 1]))
        o = jnp.concatenate(o_heads, axis=1)
        h = jnp.maximum(h + jnp.dot(o, Wo_ref[l], preferred_element_type=jnp.float32), 0.0)
    hid = jnp.maximum(jnp.dot(h, W1_ref[...], preferred_element_type=jnp.float32)
                      + b1_ref[...], 0.0)
    pred = jnp.dot(hid, W2_ref[...], preferred_element_type=jnp.float32) + b2_ref[...]
    pred_ref[0] = pred
    lp = jnp.mean((pred - x) ** 2, axis=-1, keepdims=True)
    num_ref[0] = jnp.sum(lp * m, axis=0, keepdims=True)


_mask_cache = {}


def _const_mask(B, N):
    if (B, N) not in _mask_cache:
        len_keep = int(N * (1.0 - _MASK_RATIO))
        with jax.ensure_compile_time_eval():
            noise = np.asarray(
                jax.random.uniform(jax.random.key(42), (B, N), dtype=jnp.float32))
        ids_shuffle = np.argsort(noise, axis=1, kind="stable")
        ids_restore = np.argsort(ids_shuffle, axis=1, kind="stable")
        m = np.ones((B, N), dtype=np.float32)
        m[:, :len_keep] = 0.0
        _mask_cache[(B, N)] = np.take_along_axis(m, ids_restore, axis=1)
    return _mask_cache[(B, N)]


def kernel(x, A, A_phi, mask_token, W_in, b_in, Wq, Wk, Wv, Wo, W1, b1, W2, b2):
    B, N, D = x.shape
    E = W_in.shape[1]
    mask = jnp.asarray(_const_mask(B, N))

    m3 = mask.reshape(B, N, 1)
    mt2 = mask_token.reshape(1, D)
    bin2 = b_in.reshape(1, E)
    b12 = b1.reshape(1, -1)
    b22 = b2.reshape(1, D)

    grid = (B,)
    pred, num = pl.pallas_call(
        _body,
        grid=grid,
        in_specs=[
            pl.BlockSpec((1, N, D), lambda b: (b, 0, 0)),
            pl.BlockSpec((1, N, N), lambda b: (b, 0, 0)),
            pl.BlockSpec((1, N, N), lambda b: (b, 0, 0)),
            pl.BlockSpec((1, N, 1), lambda b: (b, 0, 0)),
            pl.BlockSpec((1, D), lambda b: (0, 0)),
            pl.BlockSpec((D, E), lambda b: (0, 0)),
            pl.BlockSpec((1, E), lambda b: (0, 0)),
            pl.BlockSpec((_L, E, E), lambda b: (0, 0, 0)),
            pl.BlockSpec((_L, E, E), lambda b: (0, 0, 0)),
            pl.BlockSpec((_L, E, E), lambda b: (0, 0, 0)),
            pl.BlockSpec((_L, E, E), lambda b: (0, 0, 0)),
            pl.BlockSpec((E, 2 * E), lambda b: (0, 0)),
            pl.BlockSpec((1, 2 * E), lambda b: (0, 0)),
            pl.BlockSpec((2 * E, D), lambda b: (0, 0)),
            pl.BlockSpec((1, D), lambda b: (0, 0)),
        ],
        out_specs=[
            pl.BlockSpec((1, N, D), lambda b: (b, 0, 0)),
            pl.BlockSpec((1, 1, 1), lambda b: (b, 0, 0)),
        ],
        out_shape=[
            jax.ShapeDtypeStruct((B, N, D), jnp.float32),
            jax.ShapeDtypeStruct((B, 1, 1), jnp.float32),
        ],
        compiler_params=pltpu.CompilerParams(
            dimension_semantics=("arbitrary",)),
    )(x, A, A_phi, m3, mt2, W_in, bin2, Wq, Wk, Wv, Wo, W1, b12, W2, b22)

    loss = jnp.sum(num) / float(_const_mask(B, N).sum())
    return pred, loss, mask
